# baseline (device time: 74528 ns/iter reference)
import jax
import jax.numpy as jnp
from jax import lax
from jax.experimental import pallas as pl
from jax.experimental.pallas import tpu as pltpu

N_DEV = 4
SQ = 1024
SKV = 1024
H_LOC = 8
DH = 128
D_MODEL = 1024
D_QKV = H_LOC * DH
SCALE = 0.08838834764831843
BLK = 64
CHUNK = SQ // N_DEV


def kernel(x, Wq, K_ext, V_ext, Wo):
    def body(x_hbm, wq_hbm, k_hbm, v_hbm, wo_hbm, out_ref,
             x_v, wq_v, k_v, v_v, wo_v,
             wqb, wob, kT, vT,
             stage_ref, rs_ref, ag_ref,
             load_sems, rs_send_sems, rs_recv_sems, ag_send_sems,
             ag_recv_sems):
        my = lax.axis_index("i")

        cp_x = pltpu.make_async_copy(x_hbm.at[0], x_v, load_sems.at[0])
        cp_wq = pltpu.make_async_copy(
            wq_hbm.at[:, pl.ds(my * D_QKV, D_QKV)], wq_v, load_sems.at[1])
        cp_k = pltpu.make_async_copy(k_hbm.at[0], k_v, load_sems.at[2])
        cp_v = pltpu.make_async_copy(v_hbm.at[0], v_v, load_sems.at[3])
        cp_wo = pltpu.make_async_copy(
            wo_hbm.at[pl.ds(my * D_QKV, D_QKV), :], wo_v, load_sems.at[4])
        for cp in (cp_x, cp_wq, cp_k, cp_v, cp_wo):
            cp.start()

        barrier_sem = pltpu.get_barrier_semaphore()
        for d in range(1, N_DEV):
            pl.semaphore_signal(
                barrier_sem, inc=1,
                device_id=(lax.rem(my + d, N_DEV),),
                device_id_type=pl.DeviceIdType.MESH,
            )
        pl.semaphore_wait(barrier_sem, N_DEV - 1)

        cp_x.wait()
        cp_wq.wait()
        wqb[...] = (wq_v[...] * SCALE).astype(jnp.bfloat16)
        cp_k.wait()
        cp_v.wait()
        for h in range(H_LOC):
            kT[h] = k_v[:, h, :].astype(jnp.bfloat16)
            vT[h] = v_v[:, h, :].astype(jnp.bfloat16)
        cp_wo.wait()
        wob[...] = wo_v[...].astype(jnp.bfloat16)

        def rs_send_desc(c):
            return pltpu.make_async_remote_copy(
                src_ref=stage_ref.at[c],
                dst_ref=rs_ref.at[my],
                send_sem=rs_send_sems.at[c],
                recv_sem=rs_recv_sems.at[my],
                device_id=(c,),
                device_id_type=pl.DeviceIdType.MESH,
            )

        def rs_recv_desc(s):
            return pltpu.make_async_remote_copy(
                src_ref=stage_ref.at[s],
                dst_ref=rs_ref.at[s],
                send_sem=rs_send_sems.at[s],
                recv_sem=rs_recv_sems.at[s],
                device_id=(s,),
                device_id_type=pl.DeviceIdType.MESH,
            )

        def ag_send_desc(c, d):
            return pltpu.make_async_remote_copy(
                src_ref=ag_ref.at[c],
                dst_ref=ag_ref.at[c],
                send_sem=ag_send_sems.at[d],
                recv_sem=ag_recv_sems.at[c],
                device_id=(d,),
                device_id_type=pl.DeviceIdType.MESH,
            )

        def reduce_and_broadcast(cc):
            for s_id in range(N_DEV):
                if s_id != cc:
                    rs_recv_desc(s_id).wait_recv()
            red = (rs_ref[0].astype(jnp.float32)
                   + rs_ref[1].astype(jnp.float32)
                   + rs_ref[2].astype(jnp.float32)
                   + rs_ref[3].astype(jnp.float32))
            ag_ref[cc] = red.astype(jnp.bfloat16)
            for d in range(N_DEV):
                if d != cc:
                    ag_send_desc(cc, d).start()

        rb = lax.broadcasted_iota(jnp.int32, (CHUNK, CHUNK), 0) // BLK
        cb = lax.broadcasted_iota(jnp.int32, (CHUNK, CHUNK), 1) // BLK
        diag_bias = jnp.where(cb <= rb, 0.0, -1e9).astype(jnp.float32)

        for c in range(N_DEV):
            xc = x_v[pl.ds(c * CHUNK, CHUNK), :].astype(jnp.bfloat16)
            qc = jax.lax.dot(xc, wqb[...],
                             preferred_element_type=jnp.float32
                             ).astype(jnp.bfloat16)

            ctx_cols = []
            for h in range(H_LOC):
                qh = qc[:, h * DH:(h + 1) * DH]
                vh = vT[h, pl.ds(0, (c + 1) * CHUNK), :]
                kd = kT[h, pl.ds(c * CHUNK, CHUNK), :]
                sd = lax.dot_general(
                    qh, kd, (((1,), (1,)), ((), ())),
                    preferred_element_type=jnp.float32)
                wd = jnp.exp(sd + diag_bias)
                if c > 0:
                    kf = kT[h, pl.ds(0, c * CHUNK), :]
                    sf = lax.dot_general(
                        qh, kf, (((1,), (1,)), ((), ())),
                        preferred_element_type=jnp.float32)
                    w = jnp.concatenate([jnp.exp(sf), wd], axis=1)
                else:
                    w = wd
                denom = jnp.sum(w, axis=-1, keepdims=True)
                ctx_raw = jax.lax.dot(
                    w.astype(jnp.bfloat16), vh,
                    preferred_element_type=jnp.float32)
                ctx_cols.append(ctx_raw * (1.0 / denom))
            ctx = jnp.concatenate(ctx_cols, axis=1).astype(jnp.bfloat16)
            pc = jax.lax.dot(ctx, wob[...],
                             preferred_element_type=jnp.float32)
            pcb = pc.astype(jnp.bfloat16)

            @pl.when(c == my)
            def _():
                rs_ref[c] = pcb

            @pl.when(c != my)
            def _():
                stage_ref[c] = pcb
                rs_send_desc(c).start()

            if c >= 1:
                @pl.when(c - 1 == my)
                def _():
                    reduce_and_broadcast(c - 1)

        @pl.when(my == N_DEV - 1)
        def _():
            reduce_and_broadcast(N_DEV - 1)

        for j in range(N_DEV):
            @pl.when(j != my)
            def _():
                pltpu.make_async_remote_copy(
                    src_ref=ag_ref.at[j],
                    dst_ref=ag_ref.at[j],
                    send_sem=ag_send_sems.at[j],
                    recv_sem=ag_recv_sems.at[j],
                    device_id=(j,),
                    device_id_type=pl.DeviceIdType.MESH,
                ).wait_recv()
                out_ref[0, pl.ds(j * CHUNK, CHUNK), :] = (
                    ag_ref[j].astype(jnp.float32))

            @pl.when(j == my)
            def _():
                out_ref[0, pl.ds(j * CHUNK, CHUNK), :] = (
                    ag_ref[j].astype(jnp.float32))

        for c in range(N_DEV):
            @pl.when(c != my)
            def _():
                rs_send_desc(c).wait_send()
        for d in range(N_DEV):
            @pl.when(d != my)
            def _():
                ag_send_desc(my, d).wait_send()

    return pl.pallas_call(
        body,
        out_shape=jax.ShapeDtypeStruct((1, SQ, D_MODEL), jnp.float32),
        in_specs=[pl.BlockSpec(memory_space=pl.ANY)] * 5,
        out_specs=pl.BlockSpec(memory_space=pltpu.VMEM),
        scratch_shapes=[
            pltpu.VMEM((SQ, D_MODEL), jnp.float32),
            pltpu.VMEM((D_MODEL, D_QKV), jnp.float32),
            pltpu.VMEM((SKV, H_LOC, DH), jnp.float32),
            pltpu.VMEM((SKV, H_LOC, DH), jnp.float32),
            pltpu.VMEM((D_QKV, D_MODEL), jnp.float32),
            pltpu.VMEM((D_MODEL, D_QKV), jnp.bfloat16),
            pltpu.VMEM((D_QKV, D_MODEL), jnp.bfloat16),
            pltpu.VMEM((H_LOC, SKV, DH), jnp.bfloat16),
            pltpu.VMEM((H_LOC, SKV, DH), jnp.bfloat16),
            pltpu.VMEM((N_DEV, CHUNK, D_MODEL), jnp.bfloat16),
            pltpu.VMEM((N_DEV, CHUNK, D_MODEL), jnp.bfloat16),
            pltpu.VMEM((N_DEV, CHUNK, D_MODEL), jnp.bfloat16),
            pltpu.SemaphoreType.DMA((5,)),
            pltpu.SemaphoreType.DMA((N_DEV,)),
            pltpu.SemaphoreType.DMA((N_DEV,)),
            pltpu.SemaphoreType.DMA((N_DEV,)),
            pltpu.SemaphoreType.DMA((N_DEV,)),
        ],
        compiler_params=pltpu.CompilerParams(
            collective_id=0, vmem_limit_bytes=100 * 1024 * 1024),
    )(x, Wq, K_ext, V_ext, Wo)


# device time: 42220 ns/iter; 1.7652x vs baseline; 1.7652x over previous
import jax
import jax.numpy as jnp
from jax import lax
from jax.experimental import pallas as pl
from jax.experimental.pallas import tpu as pltpu

N_DEV = 4
SQ = 1024
SKV = 1024
H_LOC = 8
DH = 128
D_MODEL = 1024
D_QKV = H_LOC * DH
SCALE = 0.08838834764831843
BLK = 64
N_CHUNK = 8
CHUNK = SQ // N_CHUNK


def kernel(x, Wq, K_ext, V_ext, Wo):
    x2 = x[0].astype(jnp.bfloat16)
    K = jnp.transpose(K_ext[0], (1, 0, 2)).astype(jnp.bfloat16)
    V = jnp.transpose(V_ext[0], (1, 0, 2)).astype(jnp.bfloat16)

    def body(x_ref, wq_hbm, k_ref, v_ref, wo_hbm, out_ref,
             wq_v, wo_v, wqb, wob,
             stage_ref, rs_ref, ag_ref,
             load_sems, rs_send_sems, rs_recv_sems, ag_send_sems,
             ag_recv_sems):
        my = lax.axis_index("i")

        cp_wq = pltpu.make_async_copy(
            wq_hbm.at[:, pl.ds(my * D_QKV, D_QKV)], wq_v, load_sems.at[0])
        cp_wo = pltpu.make_async_copy(
            wo_hbm.at[pl.ds(my * D_QKV, D_QKV), :], wo_v, load_sems.at[1])
        cp_wq.start()
        cp_wo.start()

        barrier_sem = pltpu.get_barrier_semaphore()
        for d in range(1, N_DEV):
            pl.semaphore_signal(
                barrier_sem, inc=1,
                device_id=(lax.rem(my + d, N_DEV),),
                device_id_type=pl.DeviceIdType.MESH,
            )
        pl.semaphore_wait(barrier_sem, N_DEV - 1)

        cp_wq.wait()
        wqb[...] = (wq_v[...] * SCALE).astype(jnp.bfloat16)
        cp_wo.wait()
        wob[...] = wo_v[...].astype(jnp.bfloat16)

        def rs_send_desc(c):
            return pltpu.make_async_remote_copy(
                src_ref=stage_ref.at[c],
                dst_ref=rs_ref.at[my * 2 + c // N_DEV],
                send_sem=rs_send_sems.at[c],
                recv_sem=rs_recv_sems.at[my * 2 + c // N_DEV],
                device_id=(c % N_DEV,),
                device_id_type=pl.DeviceIdType.MESH,
            )

        def rs_recv_desc(s, half):
            return pltpu.make_async_remote_copy(
                src_ref=stage_ref.at[s],
                dst_ref=rs_ref.at[s * 2 + half],
                send_sem=rs_send_sems.at[s],
                recv_sem=rs_recv_sems.at[s * 2 + half],
                device_id=(s,),
                device_id_type=pl.DeviceIdType.MESH,
            )

        def ag_send_desc(cc, d):
            return pltpu.make_async_remote_copy(
                src_ref=ag_ref.at[cc],
                dst_ref=ag_ref.at[cc],
                send_sem=ag_send_sems.at[(cc // N_DEV) * N_DEV + d],
                recv_sem=ag_recv_sems.at[cc],
                device_id=(d,),
                device_id_type=pl.DeviceIdType.MESH,
            )

        def reduce_and_broadcast(cc):
            own = cc % N_DEV
            half = cc // N_DEV
            for s_id in range(N_DEV):
                if s_id != own:
                    rs_recv_desc(s_id, half).wait_recv()
            red = (rs_ref[0 * 2 + half].astype(jnp.float32)
                   + rs_ref[1 * 2 + half].astype(jnp.float32)
                   + rs_ref[2 * 2 + half].astype(jnp.float32)
                   + rs_ref[3 * 2 + half].astype(jnp.float32))
            ag_ref[cc] = red.astype(jnp.bfloat16)
            for d in range(N_DEV):
                if d != own:
                    ag_send_desc(cc, d).start()

        rb = lax.broadcasted_iota(jnp.int32, (CHUNK, CHUNK), 0) // BLK
        cb = lax.broadcasted_iota(jnp.int32, (CHUNK, CHUNK), 1) // BLK
        diag_bias = jnp.where(cb <= rb, 0.0, -1e9).astype(jnp.float32)

        for c in range(N_CHUNK):
            xc = x_ref[pl.ds(c * CHUNK, CHUNK), :]
            qc = jax.lax.dot(xc, wqb[...],
                             preferred_element_type=jnp.float32
                             ).astype(jnp.bfloat16)

            ctx_cols = []
            for h in range(H_LOC):
                qh = qc[:, h * DH:(h + 1) * DH]
                vh = v_ref[h, pl.ds(0, (c + 1) * CHUNK), :]
                kd = k_ref[h, pl.ds(c * CHUNK, CHUNK), :]
                sd = lax.dot_general(
                    qh, kd, (((1,), (1,)), ((), ())),
                    preferred_element_type=jnp.float32)
                wd = jnp.exp(sd + diag_bias)
                if c > 0:
                    kf = k_ref[h, pl.ds(0, c * CHUNK), :]
                    sf = lax.dot_general(
                        qh, kf, (((1,), (1,)), ((), ())),
                        preferred_element_type=jnp.float32)
                    w = jnp.concatenate([jnp.exp(sf), wd], axis=1)
                else:
                    w = wd
                denom = jnp.sum(w, axis=-1, keepdims=True)
                ctx_raw = jax.lax.dot(
                    w.astype(jnp.bfloat16), vh,
                    preferred_element_type=jnp.float32)
                ctx_cols.append(ctx_raw * (1.0 / denom))
            ctx = jnp.concatenate(ctx_cols, axis=1).astype(jnp.bfloat16)
            pc = jax.lax.dot(ctx, wob[...],
                             preferred_element_type=jnp.float32)
            pcb = pc.astype(jnp.bfloat16)

            @pl.when(c % N_DEV == my)
            def _():
                rs_ref[(c % N_DEV) * 2 + c // N_DEV] = pcb

            @pl.when(c % N_DEV != my)
            def _():
                stage_ref[c] = pcb
                rs_send_desc(c).start()

            if c >= 2:
                cc = c - 2
                @pl.when(cc % N_DEV == my)
                def _():
                    reduce_and_broadcast(cc)

        for cc in (N_CHUNK - 2, N_CHUNK - 1):
            @pl.when(cc % N_DEV == my)
            def _():
                reduce_and_broadcast(cc)

        for j in range(N_CHUNK):
            @pl.when(j % N_DEV != my)
            def _():
                pltpu.make_async_remote_copy(
                    src_ref=ag_ref.at[j],
                    dst_ref=ag_ref.at[j],
                    send_sem=ag_send_sems.at[j],
                    recv_sem=ag_recv_sems.at[j],
                    device_id=(j % N_DEV,),
                    device_id_type=pl.DeviceIdType.MESH,
                ).wait_recv()
                out_ref[0, pl.ds(j * CHUNK, CHUNK), :] = (
                    ag_ref[j].astype(jnp.float32))

            @pl.when(j % N_DEV == my)
            def _():
                out_ref[0, pl.ds(j * CHUNK, CHUNK), :] = (
                    ag_ref[j].astype(jnp.float32))

        for c in range(N_CHUNK):
            @pl.when(c % N_DEV != my)
            def _():
                rs_send_desc(c).wait_send()
        for half in range(2):
            for d in range(N_DEV):
                @pl.when(d != my)
                def _():
                    pltpu.make_async_remote_copy(
                        src_ref=ag_ref.at[half],
                        dst_ref=ag_ref.at[half],
                        send_sem=ag_send_sems.at[half * N_DEV + d],
                        recv_sem=ag_recv_sems.at[half],
                        device_id=(d,),
                        device_id_type=pl.DeviceIdType.MESH,
                    ).wait_send()

    return pl.pallas_call(
        body,
        out_shape=jax.ShapeDtypeStruct((1, SQ, D_MODEL), jnp.float32),
        in_specs=[
            pl.BlockSpec(memory_space=pltpu.VMEM),
            pl.BlockSpec(memory_space=pl.ANY),
            pl.BlockSpec(memory_space=pltpu.VMEM),
            pl.BlockSpec(memory_space=pltpu.VMEM),
            pl.BlockSpec(memory_space=pl.ANY),
        ],
        out_specs=pl.BlockSpec(memory_space=pltpu.VMEM),
        scratch_shapes=[
            pltpu.VMEM((D_MODEL, D_QKV), jnp.float32),
            pltpu.VMEM((D_QKV, D_MODEL), jnp.float32),
            pltpu.VMEM((D_MODEL, D_QKV), jnp.bfloat16),
            pltpu.VMEM((D_QKV, D_MODEL), jnp.bfloat16),
            pltpu.VMEM((N_CHUNK, CHUNK, D_MODEL), jnp.bfloat16),
            pltpu.VMEM((N_DEV * 2, CHUNK, D_MODEL), jnp.bfloat16),
            pltpu.VMEM((N_CHUNK, CHUNK, D_MODEL), jnp.bfloat16),
            pltpu.SemaphoreType.DMA((2,)),
            pltpu.SemaphoreType.DMA((N_CHUNK,)),
            pltpu.SemaphoreType.DMA((N_DEV * 2,)),
            pltpu.SemaphoreType.DMA((N_CHUNK,)),
            pltpu.SemaphoreType.DMA((N_CHUNK,)),
        ],
        compiler_params=pltpu.CompilerParams(
            collective_id=0, vmem_limit_bytes=100 * 1024 * 1024),
    )(x2, Wq, K, V, Wo)
